# baseline (speedup 1.0000x reference)
import functools

import jax
import jax.numpy as jnp
from jax import lax
from jax.experimental import pallas as pl
from jax.experimental.pallas import tpu as pltpu
from jax.experimental.pallas import tpu_sc as plsc

BATCH = 16384
NW = 32
BPW = BATCH // NW
NCHUNK = 4
CHUNK = BPW // NCHUNK

_BCAST_DNUMS = lax.GatherDimensionNumbers(
    offset_dims=(), collapsed_slice_dims=(0,), start_index_map=(0,))


def _bcast(vec, i):
    idx = jnp.full((16, 1), i, dtype=jnp.int32)
    return lax.gather(vec, idx, _BCAST_DNUMS, (1,),
                      mode=lax.GatherScatterMode.PROMISE_IN_BOUNDS)


def _body(user_h, item_h, gu_h, gi_h, mu_h, mi_h, par_h, out_h,
          idx_u, idx_i, idg_u, idg_i, par_u, par_i, mu_v, mi_v, gu_v, gi_v,
          w_v, part_v, out_v, sem):
    wid = lax.axis_index("s") * 2 + lax.axis_index("c")

    pltpu.sync_copy(user_h.at[pl.ds(wid * NCHUNK, NCHUNK)], idx_u)
    pltpu.sync_copy(item_h.at[pl.ds(wid * NCHUNK, NCHUNK)], idx_i)
    pltpu.sync_copy(par_h, w_v)

    for j in range(NCHUNK):
        for k in range(CHUNK // 16):
            s = pl.ds(k * 16, 16)
            f = pl.ds(j * CHUNK + k * 16, 16)
            u = idx_u[j, s]
            i = idx_i[j, s]
            idg_u[j, s] = jnp.right_shift(u, 1)
            idg_i[j, s] = jnp.right_shift(i, 1)
            par_u[f] = u & 1
            par_i[f] = i & 1

    copies = []
    for j in range(NCHUNK):
        d = pl.ds(j * CHUNK, CHUNK)
        copies.append(pltpu.async_copy(mu_h.at[idx_u.at[j]], mu_v.at[d], sem))
        copies.append(pltpu.async_copy(mi_h.at[idx_i.at[j]], mi_v.at[d], sem))
        copies.append(pltpu.async_copy(gu_h.at[idg_u.at[j]], gu_v.at[d], sem))
        copies.append(pltpu.async_copy(gi_h.at[idg_i.at[j]], gi_v.at[d], sem))
    for c in copies:
        c.wait()

    b1v = w_v[48, :]
    b2v = w_v[49, :]
    wlo = w_v[50, :]
    whi = w_v[51, :]
    blv = w_v[52, :]
    bl0 = blv[0]
    lane8 = jnp.arange(16, dtype=jnp.int32) & 7
    lane16 = jnp.arange(16, dtype=jnp.int32)
    zeros16 = jnp.zeros((16,), jnp.int32)

    def elem(b, carry):
        mur = mu_v[b, :]
        mir = mi_v[b, :]
        acc = b1v
        for i in range(16):
            acc = acc + _bcast(mur, i) * w_v[i, :]
        for i in range(16):
            acc = acc + _bcast(mir, i) * w_v[16 + i, :]
        h = jnp.maximum(acc, 0.0)
        acc2 = b2v
        for i in range(16):
            acc2 = acc2 + _bcast(h, i) * w_v[32 + i, :]
        h2 = jnp.maximum(acc2, 0.0)

        rowv = zeros16 + b
        pu = plsc.load_gather(par_u, [rowv])
        pi = plsc.load_gather(par_i, [rowv])
        gus = plsc.load_gather(gu_v, [rowv, lane8 + pu * 8])
        gis = plsc.load_gather(gi_v, [rowv, lane8 + pi * 8])

        part_v[b, :] = gus * gis * wlo + h2 * whi
        return carry

    lax.fori_loop(0, BPW, elem, 0)

    for g in range(BPW // 16):
        rows = lane16 + g * 16
        acc = jnp.zeros((16,), jnp.float32) + bl0
        for c in range(16):
            acc = acc + plsc.load_gather(part_v, [rows, zeros16 + c])
        out_v[pl.ds(g * 16, 16)] = 1.0 / (1.0 + jnp.exp(-acc))

    pltpu.sync_copy(out_v, out_h.at[pl.ds(wid * BPW, BPW)])


@jax.jit
def _fused(user2, item2, gu2, gi2, mu, mi, params):
    mesh = plsc.VectorSubcoreMesh(core_axis_name="c", subcore_axis_name="s")
    f = functools.partial(
        pl.kernel,
        out_type=jax.ShapeDtypeStruct((BATCH,), jnp.float32),
        mesh=mesh,
        compiler_params=pltpu.CompilerParams(
            needs_layout_passes=False, use_tc_tiling_on_sc=False),
        scratch_types=[
            pltpu.VMEM((NCHUNK, CHUNK), jnp.int32),
            pltpu.VMEM((NCHUNK, CHUNK), jnp.int32),
            pltpu.VMEM((NCHUNK, CHUNK), jnp.int32),
            pltpu.VMEM((NCHUNK, CHUNK), jnp.int32),
            pltpu.VMEM((BPW,), jnp.int32),
            pltpu.VMEM((BPW,), jnp.int32),
            pltpu.VMEM((BPW, 16), jnp.float32),
            pltpu.VMEM((BPW, 16), jnp.float32),
            pltpu.VMEM((BPW, 16), jnp.float32),
            pltpu.VMEM((BPW, 16), jnp.float32),
            pltpu.VMEM((53, 16), jnp.float32),
            pltpu.VMEM((BPW, 16), jnp.float32),
            pltpu.VMEM((BPW,), jnp.float32),
            pltpu.SemaphoreType.DMA,
        ],
    )(_body)
    return f(user2, item2, gu2, gi2, mu, mi, params)


def kernel(user, item, gmf_user_emb, gmf_item_emb, mlp_user_emb, mlp_item_emb,
           W1, b1, W2, b2, Wl, bl):
    user2 = user.reshape(128, 128)
    item2 = item.reshape(128, 128)
    gu2 = gmf_user_emb.reshape(-1, 16)
    gi2 = gmf_item_emb.reshape(-1, 16)
    wl = Wl.reshape(16)
    params = jnp.concatenate([
        W1,
        jnp.pad(W2, ((0, 0), (0, 8))),
        b1.reshape(1, 16),
        jnp.pad(b2, (0, 8)).reshape(1, 16),
        jnp.pad(wl[:8], (0, 8)).reshape(1, 16),
        jnp.pad(wl[8:], (0, 8)).reshape(1, 16),
        jnp.pad(bl, (0, 15)).reshape(1, 16),
    ], axis=0)
    return _fused(user2, item2, gu2, gi2, mlp_user_emb, mlp_item_emb, params)
